# baseline (speedup 1.0000x reference)
import functools

import jax
import jax.numpy as jnp
from jax import lax
from jax.experimental import pallas as pl
from jax.experimental.pallas import tpu as pltpu
from jax.experimental.pallas import tpu_sc as plsc

N = 10000
E = 320000
D = 128
NPAD = 10240
CH = 128
NCHUNK = E // CH
NW = 32
CPW = 80
ROWS_PER_TILE = NPAD // 16
TAU = 2.0
V_TH = 1.0
EPS = 1e-5

_mesh = plsc.VectorSubcoreMesh(
    core_axis_name="c", subcore_axis_name="s", num_cores=2, num_subcores=16)


def _zero_vmem_2d(ref, nrows):
    z = jnp.zeros((16,), jnp.float32)

    def body(i, _):
        for m in range(8):
            ref[i, pl.ds(m * 16, 16)] = z
        return 0

    lax.fori_loop(0, nrows, body, 0)


def _zero_vmem_1d(ref, n):
    z = jnp.zeros((16,), jnp.float32)

    def body(i, _):
        ref[pl.ds(i * 16, 16)] = z
        return 0

    lax.fori_loop(0, n // 16, body, 0)


@functools.partial(
    pl.kernel,
    out_type=jax.ShapeDtypeStruct((2, NPAD), jnp.float32),
    mesh=_mesh,
    scratch_types=[
        pltpu.VMEM((2, CH), jnp.int32),
        pltpu.VMEM((CH,), jnp.float32),
        pltpu.VMEM((ROWS_PER_TILE,), jnp.float32),
        pltpu.VMEM_SHARED((NPAD,), jnp.float32),
        pltpu.SemaphoreType.DMA,
        pltpu.SemaphoreType.DMA,
    ],
)
def _deg_kernel(col_hbm, out_hbm, cbuf, ones_v, zbuf, hist_sh, isem0, isem1):
    c = lax.axis_index("c")
    s = lax.axis_index("s")
    wid = s * 2 + c
    isem = (isem0, isem1)

    _zero_vmem_1d(zbuf, ROWS_PER_TILE)
    one = jnp.ones((16,), jnp.float32)
    for m in range(CH // 16):
        ones_v[pl.ds(m * 16, 16)] = one
    pltpu.sync_copy(zbuf, hist_sh.at[pl.ds(s * ROWS_PER_TILE, ROWS_PER_TILE)])
    plsc.subcore_barrier()

    def chunk(k):
        return jnp.minimum(wid + k * NW, NCHUNK)

    pltpu.async_copy(col_hbm.at[chunk(0)], cbuf.at[0], isem[0])
    pltpu.async_copy(col_hbm.at[chunk(1)], cbuf.at[1], isem[1])

    def outer(i, _):
        for b in range(2):
            k = i * 2 + b
            pltpu.make_async_copy(col_hbm.at[0], cbuf.at[b], isem[b]).wait()
            pltpu.sync_copy(ones_v, hist_sh.at[cbuf.at[b]], add=True)
            pltpu.async_copy(col_hbm.at[chunk(k + 2)], cbuf.at[b], isem[b])
        return 0

    lax.fori_loop(0, CPW // 2, outer, 0)
    pltpu.make_async_copy(col_hbm.at[0], cbuf.at[0], isem[0]).wait()
    pltpu.make_async_copy(col_hbm.at[0], cbuf.at[1], isem[1]).wait()

    plsc.subcore_barrier()
    pltpu.sync_copy(hist_sh.at[pl.ds(s * ROWS_PER_TILE, ROWS_PER_TILE)],
                    out_hbm.at[c, pl.ds(s * ROWS_PER_TILE, ROWS_PER_TILE)])


def _proj_body(x_ref, w_ref, degc_ref, g_ref, sfac_ref, msk_ref):
    deg = degc_ref[:, 0:1] + degc_ref[:, 1:2]
    dinv = jnp.where(deg > 0, 1.0 / jnp.sqrt(jnp.maximum(deg, 1e-12)), 0.0)
    sfac_ref[...] = dinv / jnp.maximum(deg, 1.0)
    msk_ref[...] = (deg > 0).astype(jnp.float32)
    h = lax.dot_general(x_ref[...], w_ref[...], (((1,), (1,)), ((), ())),
                        preferred_element_type=jnp.float32)
    g_ref[...] = h * dinv[:N, :]


_proj = pl.pallas_call(
    _proj_body,
    out_shape=(
        jax.ShapeDtypeStruct((N, D), jnp.float32),
        jax.ShapeDtypeStruct((NPAD, 1), jnp.float32),
        jax.ShapeDtypeStruct((NPAD, 1), jnp.float32),
    ),
)


@functools.partial(
    pl.kernel,
    out_type=jax.ShapeDtypeStruct((2, NPAD, D), jnp.float32),
    mesh=_mesh,
    scratch_types=[
        pltpu.VMEM((CH,), jnp.int32),
        pltpu.VMEM((CH,), jnp.int32),
        pltpu.VMEM((CH,), jnp.int32),
        pltpu.VMEM((CH,), jnp.int32),
        pltpu.VMEM((CH, D), jnp.float32),
        pltpu.VMEM((64, D), jnp.float32),
        pltpu.VMEM_SHARED((NPAD, D), jnp.float32),
        pltpu.SemaphoreType.DMA,
        pltpu.SemaphoreType.DMA,
        pltpu.SemaphoreType.DMA,
        pltpu.SemaphoreType.DMA,
        pltpu.SemaphoreType.DMA,
    ],
)
def _scatter_kernel(g_hbm, row_hbm, col_hbm, out_hbm,
                    rbuf0, rbuf1, cbuf0, cbuf1, rows, zbuf, agg_sh,
                    gsem, ir0, ir1, ic0, ic1):
    c = lax.axis_index("c")
    s = lax.axis_index("s")
    wid = s * 2 + c
    rbuf = (rbuf0, rbuf1)
    cbuf = (cbuf0, cbuf1)
    irs = (ir0, ir1)
    ics = (ic0, ic1)

    _zero_vmem_2d(zbuf, 64)
    for k in range(ROWS_PER_TILE // 64):
        pltpu.sync_copy(zbuf, agg_sh.at[pl.ds(s * ROWS_PER_TILE + k * 64, 64)])
    plsc.subcore_barrier()

    def chunk(k):
        return jnp.minimum(wid + k * NW, NCHUNK)

    for b in range(2):
        pltpu.async_copy(row_hbm.at[chunk(b)], rbuf[b], irs[b])
        pltpu.async_copy(col_hbm.at[chunk(b)], cbuf[b], ics[b])

    def outer(i, _):
        for b in range(2):
            k = i * 2 + b
            pltpu.make_async_copy(row_hbm.at[0], rbuf[b], irs[b]).wait()
            pltpu.async_copy(g_hbm.at[rbuf[b]], rows, gsem).wait()
            pltpu.make_async_copy(col_hbm.at[0], cbuf[b], ics[b]).wait()
            pltpu.sync_copy(rows, agg_sh.at[cbuf[b]], add=True)
            pltpu.async_copy(row_hbm.at[chunk(k + 2)], rbuf[b], irs[b])
            pltpu.async_copy(col_hbm.at[chunk(k + 2)], cbuf[b], ics[b])
        return 0

    lax.fori_loop(0, CPW // 2, outer, 0)
    for b in range(2):
        pltpu.make_async_copy(row_hbm.at[0], rbuf[b], irs[b]).wait()
        pltpu.make_async_copy(col_hbm.at[0], cbuf[b], ics[b]).wait()
    plsc.subcore_barrier()

    for k in range(ROWS_PER_TILE // 64):
        r0 = s * ROWS_PER_TILE + k * 64
        pltpu.sync_copy(agg_sh.at[pl.ds(r0, 64)],
                        out_hbm.at[c, pl.ds(r0, 64), :])


def _epi_body(aggp_ref, sfac_ref, msk_ref, cb_ref, bnw_ref, bnb_ref, out_ref):
    a = aggp_ref[0, :N, :] + aggp_ref[1, :N, :]
    out = a * sfac_ref[:N, :] + msk_ref[:N, :] * cb_ref[...]
    mean = jnp.mean(out, axis=0, keepdims=True)
    var = jnp.mean((out - mean) * (out - mean), axis=0, keepdims=True)
    y = (out - mean) / jnp.sqrt(var + EPS) * bnw_ref[...] + bnb_ref[...]
    out_ref[...] = (y / TAU >= V_TH).astype(jnp.float32)


_epilogue = pl.pallas_call(
    _epi_body,
    out_shape=jax.ShapeDtypeStruct((N, D), jnp.float32),
)


def kernel(x, edge_index, conv_w, conv_b, lin_res_w, lin_res_b, bn_w, bn_b):
    del lin_res_w, lin_res_b
    ei = edge_index.astype(jnp.int32)
    row_ext = jnp.concatenate(
        [ei[0].reshape(NCHUNK, CH),
         jnp.zeros((1, CH), jnp.int32)], axis=0)
    col_ext = jnp.concatenate(
        [ei[1].reshape(NCHUNK, CH),
         jnp.full((1, CH), N, jnp.int32)], axis=0)

    degp = _deg_kernel(col_ext)
    degc = jnp.transpose(degp)
    g, sfac, msk = _proj(x, conv_w, degc)
    aggp = _scatter_kernel(g, row_ext, col_ext)
    spike = _epilogue(aggp, sfac, msk,
                      conv_b.reshape(1, D),
                      bn_w.reshape(1, D), bn_b.reshape(1, D))
    return spike


# speedup vs baseline: 2.0532x; 2.0532x over previous
import functools

import jax
import jax.numpy as jnp
from jax import lax
from jax.experimental import pallas as pl
from jax.experimental.pallas import tpu as pltpu
from jax.experimental.pallas import tpu_sc as plsc

N = 10000
E = 320000
D = 128
NPAD = 10240
CH = 128
NCHUNK = E // CH
NW = 32
CPW = 80
ROWS_PER_TILE = NPAD // 16
TAU = 2.0
V_TH = 1.0
EPS = 1e-5

_mesh = plsc.VectorSubcoreMesh(
    core_axis_name="c", subcore_axis_name="s", num_cores=2, num_subcores=16)


def _zero_vmem_2d(ref, nrows):
    z = jnp.zeros((16,), jnp.float32)

    def body(i, _):
        for m in range(8):
            ref[i, pl.ds(m * 16, 16)] = z
        return 0

    lax.fori_loop(0, nrows, body, 0)


def _zero_vmem_1d(ref, n):
    z = jnp.zeros((16,), jnp.float32)

    def body(i, _):
        ref[pl.ds(i * 16, 16)] = z
        return 0

    lax.fori_loop(0, n // 16, body, 0)


@functools.partial(
    pl.kernel,
    out_type=jax.ShapeDtypeStruct((2, NPAD), jnp.float32),
    mesh=_mesh,
    scratch_types=[
        pltpu.VMEM((2, CH), jnp.int32),
        pltpu.VMEM((CH,), jnp.float32),
        pltpu.VMEM((ROWS_PER_TILE,), jnp.float32),
        pltpu.VMEM_SHARED((NPAD,), jnp.float32),
        pltpu.SemaphoreType.DMA,
        pltpu.SemaphoreType.DMA,
    ],
)
def _deg_kernel(col_hbm, out_hbm, cbuf, ones_v, zbuf, hist_sh, isem0, isem1):
    c = lax.axis_index("c")
    s = lax.axis_index("s")
    wid = s * 2 + c
    isem = (isem0, isem1)

    _zero_vmem_1d(zbuf, ROWS_PER_TILE)
    one = jnp.ones((16,), jnp.float32)
    for m in range(CH // 16):
        ones_v[pl.ds(m * 16, 16)] = one
    pltpu.sync_copy(zbuf, hist_sh.at[pl.ds(s * ROWS_PER_TILE, ROWS_PER_TILE)])
    plsc.subcore_barrier()

    def chunk(k):
        return jnp.minimum(wid + k * NW, NCHUNK)

    pltpu.async_copy(col_hbm.at[chunk(0)], cbuf.at[0], isem[0])
    pltpu.async_copy(col_hbm.at[chunk(1)], cbuf.at[1], isem[1])

    def outer(i, _):
        for b in range(2):
            k = i * 2 + b
            pltpu.make_async_copy(col_hbm.at[0], cbuf.at[b], isem[b]).wait()
            pltpu.sync_copy(ones_v, hist_sh.at[cbuf.at[b]], add=True)
            pltpu.async_copy(col_hbm.at[chunk(k + 2)], cbuf.at[b], isem[b])
        return 0

    lax.fori_loop(0, CPW // 2, outer, 0)
    pltpu.make_async_copy(col_hbm.at[0], cbuf.at[0], isem[0]).wait()
    pltpu.make_async_copy(col_hbm.at[0], cbuf.at[1], isem[1]).wait()

    plsc.subcore_barrier()
    pltpu.sync_copy(hist_sh.at[pl.ds(s * ROWS_PER_TILE, ROWS_PER_TILE)],
                    out_hbm.at[c, pl.ds(s * ROWS_PER_TILE, ROWS_PER_TILE)])


def _proj_body(x_ref, w_ref, degc_ref, g_ref, sfac_ref, msk_ref):
    deg = degc_ref[:, 0:1] + degc_ref[:, 1:2]
    dinv = jnp.where(deg > 0, 1.0 / jnp.sqrt(jnp.maximum(deg, 1e-12)), 0.0)
    sfac_ref[...] = dinv / jnp.maximum(deg, 1.0)
    msk_ref[...] = (deg > 0).astype(jnp.float32)
    h = lax.dot_general(x_ref[...], w_ref[...], (((1,), (1,)), ((), ())),
                        preferred_element_type=jnp.float32)
    g_ref[...] = h * dinv[:N, :]


_proj = pl.pallas_call(
    _proj_body,
    out_shape=(
        jax.ShapeDtypeStruct((N, D), jnp.float32),
        jax.ShapeDtypeStruct((NPAD, 1), jnp.float32),
        jax.ShapeDtypeStruct((NPAD, 1), jnp.float32),
    ),
)


@functools.partial(
    pl.kernel,
    out_type=jax.ShapeDtypeStruct((2, NPAD, D), jnp.float32),
    mesh=_mesh,
    scratch_types=[
        pltpu.VMEM((CH,), jnp.int32),
        pltpu.VMEM((CH,), jnp.int32),
        pltpu.VMEM((CH,), jnp.int32),
        pltpu.VMEM((CH,), jnp.int32),
        pltpu.VMEM((CH, D), jnp.float32),
        pltpu.VMEM((64, D), jnp.float32),
        pltpu.VMEM_SHARED((NPAD, D), jnp.float32),
        pltpu.SemaphoreType.DMA,
        pltpu.SemaphoreType.DMA,
        pltpu.SemaphoreType.DMA,
        pltpu.SemaphoreType.DMA,
        pltpu.SemaphoreType.DMA,
    ],
)
def _scatter_kernel(g_hbm, row_hbm, col_hbm, out_hbm,
                    rbuf0, rbuf1, cbuf0, cbuf1, rows, zbuf, agg_sh,
                    gsem, ir0, ir1, ic0, ic1):
    c = lax.axis_index("c")
    s = lax.axis_index("s")
    wid = s * 2 + c
    rbuf = (rbuf0, rbuf1)
    cbuf = (cbuf0, cbuf1)
    irs = (ir0, ir1)
    ics = (ic0, ic1)

    _zero_vmem_2d(zbuf, 64)
    for k in range(ROWS_PER_TILE // 64):
        pltpu.sync_copy(zbuf, agg_sh.at[pl.ds(s * ROWS_PER_TILE + k * 64, 64)])
    plsc.subcore_barrier()

    def chunk(k):
        return jnp.minimum(wid + k * NW, NCHUNK)

    for b in range(2):
        pltpu.async_copy(row_hbm.at[chunk(b)], rbuf[b], irs[b])
        pltpu.async_copy(col_hbm.at[chunk(b)], cbuf[b], ics[b])

    def outer(i, _):
        for b in range(2):
            k = i * 2 + b
            pltpu.make_async_copy(row_hbm.at[0], rbuf[b], irs[b]).wait()
            pltpu.async_copy(g_hbm.at[rbuf[b]], rows, gsem).wait()
            pltpu.make_async_copy(col_hbm.at[0], cbuf[b], ics[b]).wait()
            pltpu.sync_copy(rows, agg_sh.at[cbuf[b]], add=True)
            pltpu.async_copy(row_hbm.at[chunk(k + 2)], rbuf[b], irs[b])
            pltpu.async_copy(col_hbm.at[chunk(k + 2)], cbuf[b], ics[b])
        return 0

    lax.fori_loop(0, CPW // 2, outer, 0)
    for b in range(2):
        pltpu.make_async_copy(row_hbm.at[0], rbuf[b], irs[b]).wait()
        pltpu.make_async_copy(col_hbm.at[0], cbuf[b], ics[b]).wait()
    plsc.subcore_barrier()

    for k in range(ROWS_PER_TILE // 64):
        r0 = s * ROWS_PER_TILE + k * 64
        pltpu.sync_copy(agg_sh.at[pl.ds(r0, 64)],
                        out_hbm.at[c, pl.ds(r0, 64), :])


def _epi_body(aggp_ref, sfac_ref, msk_ref, cb_ref, bnw_ref, bnb_ref, out_ref):
    a = aggp_ref[0, :N, :] + aggp_ref[1, :N, :]
    out = a * sfac_ref[:N, :] + msk_ref[:N, :] * cb_ref[...]
    mean = jnp.mean(out, axis=0, keepdims=True)
    var = jnp.mean((out - mean) * (out - mean), axis=0, keepdims=True)
    y = (out - mean) / jnp.sqrt(var + EPS) * bnw_ref[...] + bnb_ref[...]
    out_ref[...] = (y / TAU >= V_TH).astype(jnp.float32)


_epilogue = pl.pallas_call(
    _epi_body,
    out_shape=jax.ShapeDtypeStruct((N, D), jnp.float32),
)


def kernel(x, edge_index, conv_w, conv_b, lin_res_w, lin_res_b, bn_w, bn_b):
    del lin_res_w, lin_res_b
    ei = edge_index.astype(jnp.int32)
    lanes = jnp.arange(CH, dtype=jnp.int32)[None, :]
    row_ext = jnp.concatenate(
        [ei[0].reshape(NCHUNK, CH), lanes], axis=0)
    col_ext = jnp.concatenate(
        [ei[1].reshape(NCHUNK, CH), N + lanes], axis=0)

    degp = _deg_kernel(col_ext)
    degc = jnp.transpose(degp)
    g, sfac, msk = _proj(x, conv_w, degc)
    aggp = _scatter_kernel(g, row_ext, col_ext)
    spike = _epilogue(aggp, sfac, msk,
                      conv_b.reshape(1, D),
                      bn_w.reshape(1, D), bn_b.reshape(1, D))
    return spike


# speedup vs baseline: 2.5343x; 1.2343x over previous
import functools

import jax
import jax.numpy as jnp
from jax import lax
from jax.experimental import pallas as pl
from jax.experimental.pallas import tpu as pltpu
from jax.experimental.pallas import tpu_sc as plsc

N = 10000
E = 320000
D = 128
NPAD = 10240
CH = 128
NCHUNK = E // CH
NW = 32
CPW = 80
ROWS_PER_TILE = NPAD // 16
TAU = 2.0
V_TH = 1.0
EPS = 1e-5

_mesh = plsc.VectorSubcoreMesh(
    core_axis_name="c", subcore_axis_name="s", num_cores=2, num_subcores=16)


def _zero_vmem_2d(ref, nrows):
    z = jnp.zeros((16,), jnp.float32)

    def body(i, _):
        for m in range(8):
            ref[i, pl.ds(m * 16, 16)] = z
        return 0

    lax.fori_loop(0, nrows, body, 0)


def _zero_vmem_1d(ref, n):
    z = jnp.zeros((16,), jnp.float32)

    def body(i, _):
        ref[pl.ds(i * 16, 16)] = z
        return 0

    lax.fori_loop(0, n // 16, body, 0)


@functools.partial(
    pl.kernel,
    out_type=jax.ShapeDtypeStruct((2, NPAD), jnp.float32),
    mesh=_mesh,
    scratch_types=[
        pltpu.VMEM((2, CH), jnp.int32),
        pltpu.VMEM((CH,), jnp.float32),
        pltpu.VMEM((ROWS_PER_TILE,), jnp.float32),
        pltpu.VMEM_SHARED((NPAD,), jnp.float32),
        pltpu.SemaphoreType.DMA,
        pltpu.SemaphoreType.DMA,
    ],
)
def _deg_kernel(col_hbm, out_hbm, cbuf, ones_v, zbuf, hist_sh, isem0, isem1):
    c = lax.axis_index("c")
    s = lax.axis_index("s")
    wid = s * 2 + c
    isem = (isem0, isem1)

    _zero_vmem_1d(zbuf, ROWS_PER_TILE)
    one = jnp.ones((16,), jnp.float32)
    for m in range(CH // 16):
        ones_v[pl.ds(m * 16, 16)] = one
    pltpu.sync_copy(zbuf, hist_sh.at[pl.ds(s * ROWS_PER_TILE, ROWS_PER_TILE)])
    plsc.subcore_barrier()

    def chunk(k):
        return jnp.minimum(wid + k * NW, NCHUNK)

    pltpu.async_copy(col_hbm.at[chunk(0)], cbuf.at[0], isem[0])
    pltpu.async_copy(col_hbm.at[chunk(1)], cbuf.at[1], isem[1])

    def outer(i, _):
        for b in range(2):
            k = i * 2 + b
            pltpu.make_async_copy(col_hbm.at[0], cbuf.at[b], isem[b]).wait()
            pltpu.sync_copy(ones_v, hist_sh.at[cbuf.at[b]], add=True)
            pltpu.async_copy(col_hbm.at[chunk(k + 2)], cbuf.at[b], isem[b])
        return 0

    lax.fori_loop(0, CPW // 2, outer, 0)
    pltpu.make_async_copy(col_hbm.at[0], cbuf.at[0], isem[0]).wait()
    pltpu.make_async_copy(col_hbm.at[0], cbuf.at[1], isem[1]).wait()

    plsc.subcore_barrier()
    pltpu.sync_copy(hist_sh.at[pl.ds(s * ROWS_PER_TILE, ROWS_PER_TILE)],
                    out_hbm.at[c, pl.ds(s * ROWS_PER_TILE, ROWS_PER_TILE)])


def _proj_body(x_ref, w_ref, degc_ref, g_ref, sfac_ref, msk_ref):
    deg = degc_ref[:, 0:1] + degc_ref[:, 1:2]
    dinv = jnp.where(deg > 0, 1.0 / jnp.sqrt(jnp.maximum(deg, 1e-12)), 0.0)
    sfac_ref[...] = dinv / jnp.maximum(deg, 1.0)
    msk_ref[...] = (deg > 0).astype(jnp.float32)
    h = lax.dot_general(x_ref[...], w_ref[...], (((1,), (1,)), ((), ())),
                        preferred_element_type=jnp.float32)
    g_ref[...] = h * dinv[:N, :]


_proj = pl.pallas_call(
    _proj_body,
    out_shape=(
        jax.ShapeDtypeStruct((N, D), jnp.float32),
        jax.ShapeDtypeStruct((NPAD, 1), jnp.float32),
        jax.ShapeDtypeStruct((NPAD, 1), jnp.float32),
    ),
)


@functools.partial(
    pl.kernel,
    out_type=jax.ShapeDtypeStruct((2, NPAD, D), jnp.float32),
    mesh=_mesh,
    scratch_types=[
        pltpu.VMEM((CH,), jnp.int32),
        pltpu.VMEM((CH,), jnp.int32),
        pltpu.VMEM((CH,), jnp.int32),
        pltpu.VMEM((CH,), jnp.int32),
        pltpu.VMEM((2, CH, D), jnp.float32),
        pltpu.VMEM((64, D), jnp.float32),
        pltpu.VMEM_SHARED((NPAD, D), jnp.float32),
        pltpu.SemaphoreType.DMA,
        pltpu.SemaphoreType.DMA,
        pltpu.SemaphoreType.DMA,
        pltpu.SemaphoreType.DMA,
        pltpu.SemaphoreType.DMA,
        pltpu.SemaphoreType.DMA,
    ],
)
def _scatter_kernel(g_hbm, row_hbm, col_hbm, out_hbm,
                    rbuf0, rbuf1, cbuf0, cbuf1, rows, zbuf, agg_sh,
                    gsem0, gsem1, ir0, ir1, ic0, ic1):
    c = lax.axis_index("c")
    s = lax.axis_index("s")
    wid = s * 2 + c
    rbuf = (rbuf0, rbuf1)
    cbuf = (cbuf0, cbuf1)
    gsem = (gsem0, gsem1)
    irs = (ir0, ir1)
    ics = (ic0, ic1)

    _zero_vmem_2d(zbuf, 64)
    for k in range(ROWS_PER_TILE // 64):
        pltpu.sync_copy(zbuf, agg_sh.at[pl.ds(s * ROWS_PER_TILE + k * 64, 64)])
    plsc.subcore_barrier()

    def chunk(k):
        return jnp.minimum(wid + k * NW, NCHUNK)

    for b in range(2):
        pltpu.async_copy(row_hbm.at[chunk(b)], rbuf[b], irs[b])
        pltpu.async_copy(col_hbm.at[chunk(b)], cbuf[b], ics[b])
    pltpu.make_async_copy(row_hbm.at[0], rbuf[0], irs[0]).wait()
    pltpu.async_copy(g_hbm.at[rbuf[0]], rows.at[0], gsem[0])

    def outer(i, _):
        for b in range(2):
            k = i * 2 + b
            bn = (b + 1) % 2
            pltpu.make_async_copy(row_hbm.at[0], rbuf[bn], irs[bn]).wait()
            pltpu.async_copy(g_hbm.at[rbuf[bn]], rows.at[bn], gsem[bn])
            pltpu.make_async_copy(
                g_hbm.at[rbuf[b]], rows.at[b], gsem[b]).wait()
            pltpu.make_async_copy(col_hbm.at[0], cbuf[b], ics[b]).wait()
            pltpu.sync_copy(rows.at[b], agg_sh.at[cbuf[b]], add=True)
            pltpu.async_copy(row_hbm.at[chunk(k + 2)], rbuf[b], irs[b])
            pltpu.async_copy(col_hbm.at[chunk(k + 2)], cbuf[b], ics[b])
        return 0

    lax.fori_loop(0, CPW // 2, outer, 0)
    pltpu.make_async_copy(g_hbm.at[rbuf[0]], rows.at[0], gsem[0]).wait()
    pltpu.make_async_copy(col_hbm.at[0], cbuf[0], ics[0]).wait()
    pltpu.make_async_copy(row_hbm.at[0], rbuf[1], irs[1]).wait()
    pltpu.make_async_copy(col_hbm.at[0], cbuf[1], ics[1]).wait()
    plsc.subcore_barrier()

    for k in range(ROWS_PER_TILE // 64):
        r0 = s * ROWS_PER_TILE + k * 64
        pltpu.sync_copy(agg_sh.at[pl.ds(r0, 64)],
                        out_hbm.at[c, pl.ds(r0, 64), :])


def _epi_body(aggp_ref, sfac_ref, msk_ref, cb_ref, bnw_ref, bnb_ref, out_ref):
    a = aggp_ref[0, :N, :] + aggp_ref[1, :N, :]
    out = a * sfac_ref[:N, :] + msk_ref[:N, :] * cb_ref[...]
    mean = jnp.mean(out, axis=0, keepdims=True)
    var = jnp.mean((out - mean) * (out - mean), axis=0, keepdims=True)
    y = (out - mean) / jnp.sqrt(var + EPS) * bnw_ref[...] + bnb_ref[...]
    out_ref[...] = (y / TAU >= V_TH).astype(jnp.float32)


_epilogue = pl.pallas_call(
    _epi_body,
    out_shape=jax.ShapeDtypeStruct((N, D), jnp.float32),
)


def kernel(x, edge_index, conv_w, conv_b, lin_res_w, lin_res_b, bn_w, bn_b):
    del lin_res_w, lin_res_b
    ei = edge_index.astype(jnp.int32)
    lanes = jnp.arange(CH, dtype=jnp.int32)[None, :]
    row_ext = jnp.concatenate(
        [ei[0].reshape(NCHUNK, CH), lanes], axis=0)
    col_ext = jnp.concatenate(
        [ei[1].reshape(NCHUNK, CH), N + lanes], axis=0)

    degp = _deg_kernel(col_ext)
    degc = jnp.transpose(degp)
    g, sfac, msk = _proj(x, conv_w, degc)
    aggp = _scatter_kernel(g, row_ext, col_ext)
    spike = _epilogue(aggp, sfac, msk,
                      conv_b.reshape(1, D),
                      bn_w.reshape(1, D), bn_b.reshape(1, D))
    return spike


# speedup vs baseline: 2.7529x; 1.0863x over previous
import functools

import jax
import jax.numpy as jnp
from jax import lax
from jax.experimental import pallas as pl
from jax.experimental.pallas import tpu as pltpu
from jax.experimental.pallas import tpu_sc as plsc

N = 10000
E = 320000
D = 128
NPAD = 10240
CH = 128
NCHUNK = E // CH
NW = 32
CPW = 80
ROWS_PER_TILE = NPAD // 16
TAU = 2.0
V_TH = 1.0
EPS = 1e-5

_mesh = plsc.VectorSubcoreMesh(
    core_axis_name="c", subcore_axis_name="s", num_cores=2, num_subcores=16)


def _zero_vmem_2d(ref, nrows):
    z = jnp.zeros((16,), jnp.float32)

    def body(i, _):
        for m in range(8):
            ref[i, pl.ds(m * 16, 16)] = z
        return 0

    lax.fori_loop(0, nrows, body, 0)


def _zero_vmem_1d(ref, n):
    z = jnp.zeros((16,), jnp.float32)

    def body(i, _):
        ref[pl.ds(i * 16, 16)] = z
        return 0

    lax.fori_loop(0, n // 16, body, 0)


@functools.partial(
    pl.kernel,
    out_type=jax.ShapeDtypeStruct((2, NPAD), jnp.float32),
    mesh=_mesh,
    scratch_types=[
        pltpu.VMEM((2, CH), jnp.int32),
        pltpu.VMEM((CH,), jnp.float32),
        pltpu.VMEM((ROWS_PER_TILE,), jnp.float32),
        pltpu.VMEM_SHARED((NPAD,), jnp.float32),
        pltpu.SemaphoreType.DMA,
        pltpu.SemaphoreType.DMA,
    ],
)
def _deg_kernel(col_hbm, out_hbm, cbuf, ones_v, zbuf, hist_sh, isem0, isem1):
    c = lax.axis_index("c")
    s = lax.axis_index("s")
    wid = s * 2 + c
    isem = (isem0, isem1)

    _zero_vmem_1d(zbuf, ROWS_PER_TILE)
    one = jnp.ones((16,), jnp.float32)
    for m in range(CH // 16):
        ones_v[pl.ds(m * 16, 16)] = one
    pltpu.sync_copy(zbuf, hist_sh.at[pl.ds(s * ROWS_PER_TILE, ROWS_PER_TILE)])
    plsc.subcore_barrier()

    def chunk(k):
        return jnp.minimum(wid + k * NW, NCHUNK)

    pltpu.async_copy(col_hbm.at[chunk(0)], cbuf.at[0], isem[0])
    pltpu.async_copy(col_hbm.at[chunk(1)], cbuf.at[1], isem[1])

    def outer(i, _):
        for b in range(2):
            k = i * 2 + b
            pltpu.make_async_copy(col_hbm.at[0], cbuf.at[b], isem[b]).wait()
            pltpu.sync_copy(ones_v, hist_sh.at[cbuf.at[b]], add=True)
            pltpu.async_copy(col_hbm.at[chunk(k + 2)], cbuf.at[b], isem[b])
        return 0

    lax.fori_loop(0, CPW // 2, outer, 0)
    pltpu.make_async_copy(col_hbm.at[0], cbuf.at[0], isem[0]).wait()
    pltpu.make_async_copy(col_hbm.at[0], cbuf.at[1], isem[1]).wait()

    plsc.subcore_barrier()
    pltpu.sync_copy(hist_sh.at[pl.ds(s * ROWS_PER_TILE, ROWS_PER_TILE)],
                    out_hbm.at[c, pl.ds(s * ROWS_PER_TILE, ROWS_PER_TILE)])


def _proj_body(x_ref, w_ref, degc_ref, g_ref, sfac_ref, msk_ref):
    deg = degc_ref[:, 0:1] + degc_ref[:, 1:2]
    dinv = jnp.where(deg > 0, 1.0 / jnp.sqrt(jnp.maximum(deg, 1e-12)), 0.0)
    sfac_ref[...] = dinv / jnp.maximum(deg, 1.0)
    msk_ref[...] = (deg > 0).astype(jnp.float32)
    h = lax.dot_general(x_ref[...], w_ref[...], (((1,), (1,)), ((), ())),
                        preferred_element_type=jnp.float32)
    g_ref[...] = h * dinv[:N, :]


_proj = pl.pallas_call(
    _proj_body,
    out_shape=(
        jax.ShapeDtypeStruct((N, D), jnp.float32),
        jax.ShapeDtypeStruct((NPAD, 1), jnp.float32),
        jax.ShapeDtypeStruct((NPAD, 1), jnp.float32),
    ),
)


@functools.partial(
    pl.kernel,
    out_type=jax.ShapeDtypeStruct((2, NPAD, D), jnp.float32),
    mesh=_mesh,
    scratch_types=[
        pltpu.VMEM((CH,), jnp.int32),
        pltpu.VMEM((CH,), jnp.int32),
        pltpu.VMEM((CH,), jnp.int32),
        pltpu.VMEM((CH,), jnp.int32),
        pltpu.VMEM((CH,), jnp.int32),
        pltpu.VMEM((CH,), jnp.int32),
        pltpu.VMEM((2, CH, D), jnp.float32),
        pltpu.VMEM((64, D), jnp.float32),
        pltpu.VMEM_SHARED((NPAD, D), jnp.float32),
        pltpu.SemaphoreType.DMA,
        pltpu.SemaphoreType.DMA,
        pltpu.SemaphoreType.DMA,
        pltpu.SemaphoreType.DMA,
        pltpu.SemaphoreType.DMA,
        pltpu.SemaphoreType.DMA,
        pltpu.SemaphoreType.DMA,
        pltpu.SemaphoreType.DMA,
        pltpu.SemaphoreType.DMA,
        pltpu.SemaphoreType.DMA,
    ],
)
def _scatter_kernel(g_hbm, row_hbm, col_hbm, out_hbm,
                    rbuf0, rbuf1, rbuf2, cbuf0, cbuf1, cbuf2, rows, zbuf,
                    agg_sh, gsem0, gsem1, ssem0, ssem1,
                    ir0, ir1, ir2, ic0, ic1, ic2):
    c = lax.axis_index("c")
    s = lax.axis_index("s")
    wid = s * 2 + c
    rbuf = (rbuf0, rbuf1, rbuf2)
    cbuf = (cbuf0, cbuf1, cbuf2)
    gsem = (gsem0, gsem1)
    ssem = (ssem0, ssem1)
    irs = (ir0, ir1, ir2)
    ics = (ic0, ic1, ic2)

    _zero_vmem_2d(zbuf, 64)
    for k in range(ROWS_PER_TILE // 64):
        pltpu.sync_copy(zbuf, agg_sh.at[pl.ds(s * ROWS_PER_TILE + k * 64, 64)])
    plsc.subcore_barrier()

    def chunk(k):
        return jnp.minimum(wid + k * NW, NCHUNK)

    def gather(k_rows_slot, idx_slot):
        pltpu.async_copy(g_hbm.at[rbuf[idx_slot]], rows.at[k_rows_slot],
                         gsem[k_rows_slot])

    def scatter_start(rows_slot, idx_slot):
        pltpu.async_copy(rows.at[rows_slot], agg_sh.at[cbuf[idx_slot]],
                         ssem[rows_slot], add=True)

    def scatter_wait(rows_slot, idx_slot):
        pltpu.make_async_copy(rows.at[rows_slot], agg_sh.at[cbuf[idx_slot]],
                              ssem[rows_slot]).wait()

    for b in range(2):
        pltpu.async_copy(row_hbm.at[chunk(b)], rbuf[b], irs[b])
        pltpu.async_copy(col_hbm.at[chunk(b)], cbuf[b], ics[b])
    pltpu.make_async_copy(row_hbm.at[0], rbuf[0], irs[0]).wait()
    gather(0, 0)

    def step(k, phase, peeled_first=False):
        br = phase % 2
        brn = (br + 1) % 2
        bi = phase % 3
        bi1 = (phase + 1) % 3
        bi2 = (phase + 2) % 3
        pltpu.make_async_copy(row_hbm.at[0], rbuf[bi1], irs[bi1]).wait()
        if not peeled_first:
            scatter_wait(brn, bi2)
        gather(brn, bi1)
        pltpu.async_copy(row_hbm.at[chunk(k + 2)], rbuf[bi2], irs[bi2])
        pltpu.async_copy(col_hbm.at[chunk(k + 2)], cbuf[bi2], ics[bi2])
        pltpu.make_async_copy(
            g_hbm.at[rbuf[bi]], rows.at[br], gsem[br]).wait()
        pltpu.make_async_copy(col_hbm.at[0], cbuf[bi], ics[bi]).wait()
        scatter_start(br, bi)

    step(0, 0, peeled_first=True)

    def outer(i, _):
        for t in range(6):
            step(1 + i * 6 + t, 1 + t)
        return 0

    lax.fori_loop(0, (CPW - 2) // 6, outer, 0)
    step(CPW - 1, CPW - 1)
    scatter_wait((CPW - 1) % 2, (CPW - 1) % 3)
    pltpu.make_async_copy(
        g_hbm.at[rbuf[CPW % 3]], rows.at[CPW % 2], gsem[CPW % 2]).wait()
    pltpu.make_async_copy(
        row_hbm.at[0], rbuf[(CPW + 1) % 3], irs[(CPW + 1) % 3]).wait()
    pltpu.make_async_copy(
        col_hbm.at[0], cbuf[CPW % 3], ics[CPW % 3]).wait()
    pltpu.make_async_copy(
        col_hbm.at[0], cbuf[(CPW + 1) % 3], ics[(CPW + 1) % 3]).wait()
    plsc.subcore_barrier()

    for k in range(ROWS_PER_TILE // 128):
        r0 = s * ROWS_PER_TILE + k * 128
        pltpu.sync_copy(agg_sh.at[pl.ds(r0, 128)],
                        out_hbm.at[c, pl.ds(r0, 128), :])


def _epi_body(aggp_ref, sfac_ref, msk_ref, cb_ref, bnw_ref, bnb_ref, out_ref):
    a = aggp_ref[0, :N, :] + aggp_ref[1, :N, :]
    out = a * sfac_ref[:N, :] + msk_ref[:N, :] * cb_ref[...]
    mean = jnp.mean(out, axis=0, keepdims=True)
    var = jnp.mean((out - mean) * (out - mean), axis=0, keepdims=True)
    y = (out - mean) / jnp.sqrt(var + EPS) * bnw_ref[...] + bnb_ref[...]
    out_ref[...] = (y / TAU >= V_TH).astype(jnp.float32)


_epilogue = pl.pallas_call(
    _epi_body,
    out_shape=jax.ShapeDtypeStruct((N, D), jnp.float32),
)


def kernel(x, edge_index, conv_w, conv_b, lin_res_w, lin_res_b, bn_w, bn_b):
    del lin_res_w, lin_res_b
    ei = edge_index.astype(jnp.int32)
    lanes = jnp.arange(CH, dtype=jnp.int32)[None, :]
    row_ext = jnp.concatenate(
        [ei[0].reshape(NCHUNK, CH), lanes], axis=0)
    col_ext = jnp.concatenate(
        [ei[1].reshape(NCHUNK, CH), N + lanes], axis=0)

    degp = _deg_kernel(col_ext)
    degc = jnp.transpose(degp)
    g, sfac, msk = _proj(x, conv_w, degc)
    aggp = _scatter_kernel(g, row_ext, col_ext)
    spike = _epilogue(aggp, sfac, msk,
                      conv_b.reshape(1, D),
                      bn_w.reshape(1, D), bn_b.reshape(1, D))
    return spike


# speedup vs baseline: 2.8264x; 1.0267x over previous
import functools

import jax
import jax.numpy as jnp
from jax import lax
from jax.experimental import pallas as pl
from jax.experimental.pallas import tpu as pltpu
from jax.experimental.pallas import tpu_sc as plsc

N = 10000
E = 320000
D = 128
NPAD = 10240
CH = 128
NCHUNK = E // CH
NW = 32
CPW = 80
ROWS_PER_TILE = NPAD // 16
TAU = 2.0
V_TH = 1.0
EPS = 1e-5

_mesh = plsc.VectorSubcoreMesh(
    core_axis_name="c", subcore_axis_name="s", num_cores=2, num_subcores=16)


def _zero_vmem_2d(ref, nrows):
    z = jnp.zeros((16,), jnp.float32)

    def body(i, _):
        for m in range(8):
            ref[i, pl.ds(m * 16, 16)] = z
        return 0

    lax.fori_loop(0, nrows, body, 0)


def _zero_vmem_1d(ref, n):
    z = jnp.zeros((16,), jnp.float32)

    def body(i, _):
        ref[pl.ds(i * 16, 16)] = z
        return 0

    lax.fori_loop(0, n // 16, body, 0)


@functools.partial(
    pl.kernel,
    out_type=jax.ShapeDtypeStruct((2, NPAD), jnp.float32),
    mesh=_mesh,
    scratch_types=[
        pltpu.VMEM((4, CH), jnp.int32),
        pltpu.VMEM((CH,), jnp.float32),
        pltpu.VMEM((ROWS_PER_TILE,), jnp.float32),
        pltpu.VMEM_SHARED((NPAD,), jnp.float32),
        pltpu.SemaphoreType.DMA,
        pltpu.SemaphoreType.DMA,
        pltpu.SemaphoreType.DMA,
        pltpu.SemaphoreType.DMA,
        pltpu.SemaphoreType.DMA,
        pltpu.SemaphoreType.DMA,
        pltpu.SemaphoreType.DMA,
        pltpu.SemaphoreType.DMA,
    ],
)
def _deg_kernel(col_hbm, out_hbm, cbuf, ones_v, zbuf, hist_sh,
                isem0, isem1, isem2, isem3, ssem0, ssem1, ssem2, ssem3):
    c = lax.axis_index("c")
    s = lax.axis_index("s")
    wid = s * 2 + c
    isem = (isem0, isem1, isem2, isem3)
    ssem = (ssem0, ssem1, ssem2, ssem3)

    _zero_vmem_1d(zbuf, ROWS_PER_TILE)
    one = jnp.ones((16,), jnp.float32)
    for m in range(CH // 16):
        ones_v[pl.ds(m * 16, 16)] = one
    pltpu.sync_copy(zbuf, hist_sh.at[pl.ds(s * ROWS_PER_TILE, ROWS_PER_TILE)])
    plsc.subcore_barrier()

    def chunk(k):
        return jnp.minimum(wid + k * NW, NCHUNK)

    def swait(b):
        pltpu.make_async_copy(ones_v, hist_sh.at[cbuf.at[b]], ssem[b]).wait()

    pltpu.async_copy(col_hbm.at[chunk(0)], cbuf.at[0], isem[0])
    pltpu.async_copy(col_hbm.at[chunk(1)], cbuf.at[1], isem[1])

    def step(k, phase, first=False):
        b = phase % 4
        b2 = (phase + 2) % 4
        pltpu.make_async_copy(col_hbm.at[0], cbuf.at[b], isem[b]).wait()
        pltpu.async_copy(ones_v, hist_sh.at[cbuf.at[b]], ssem[b], add=True)
        if not first:
            swait(b2)
        pltpu.async_copy(col_hbm.at[chunk(k + 2)], cbuf.at[b2], isem[b2])

    step(0, 0, first=True)
    step(1, 1, first=True)

    def outer(i, _):
        for t in range(4):
            step(2 + i * 4 + t, 2 + t)
        return 0

    lax.fori_loop(0, (CPW - 4) // 4, outer, 0)
    step(CPW - 2, CPW - 2)
    step(CPW - 1, CPW - 1)
    swait((CPW - 2) % 4)
    swait((CPW - 1) % 4)
    pltpu.make_async_copy(col_hbm.at[0], cbuf.at[CPW % 4], isem[CPW % 4]).wait()
    pltpu.make_async_copy(
        col_hbm.at[0], cbuf.at[(CPW + 1) % 4], isem[(CPW + 1) % 4]).wait()

    plsc.subcore_barrier()
    pltpu.sync_copy(hist_sh.at[pl.ds(s * ROWS_PER_TILE, ROWS_PER_TILE)],
                    out_hbm.at[c, pl.ds(s * ROWS_PER_TILE, ROWS_PER_TILE)])


def _proj_body(x_ref, w_ref, degc_ref, g_ref, sfac_ref, msk_ref):
    deg = degc_ref[:, 0:1] + degc_ref[:, 1:2]
    dinv = jnp.where(deg > 0, 1.0 / jnp.sqrt(jnp.maximum(deg, 1e-12)), 0.0)
    sfac_ref[...] = dinv / jnp.maximum(deg, 1.0)
    msk_ref[...] = (deg > 0).astype(jnp.float32)
    h = lax.dot_general(x_ref[...], w_ref[...], (((1,), (1,)), ((), ())),
                        preferred_element_type=jnp.float32)
    g_ref[...] = h * dinv[:N, :]


_proj = pl.pallas_call(
    _proj_body,
    out_shape=(
        jax.ShapeDtypeStruct((N, D), jnp.float32),
        jax.ShapeDtypeStruct((NPAD, 1), jnp.float32),
        jax.ShapeDtypeStruct((NPAD, 1), jnp.float32),
    ),
)


@functools.partial(
    pl.kernel,
    out_type=jax.ShapeDtypeStruct((2, NPAD, D), jnp.float32),
    mesh=_mesh,
    scratch_types=[
        pltpu.VMEM((2, CH), jnp.int32),
        pltpu.VMEM((2, CH), jnp.int32),
        pltpu.VMEM((2, CH), jnp.int32),
        pltpu.VMEM((2, CH, D), jnp.float32),
        pltpu.VMEM((64, D), jnp.float32),
        pltpu.VMEM_SHARED((NPAD, D), jnp.float32),
        pltpu.SemaphoreType.DMA,
        pltpu.SemaphoreType.DMA,
        pltpu.SemaphoreType.DMA,
        pltpu.SemaphoreType.DMA,
        pltpu.SemaphoreType.DMA,
        pltpu.SemaphoreType.DMA,
        pltpu.SemaphoreType.DMA,
    ],
)
def _scatter_kernel(g_hbm, eidx_hbm, out_hbm,
                    ebuf0, ebuf1, ebuf2, rows, zbuf,
                    agg_sh, gsem0, gsem1, ssem0, ssem1, is0, is1, is2):
    c = lax.axis_index("c")
    s = lax.axis_index("s")
    wid = s * 2 + c
    ebuf = (ebuf0, ebuf1, ebuf2)
    gsem = (gsem0, gsem1)
    ssem = (ssem0, ssem1)
    isem = (is0, is1, is2)

    _zero_vmem_2d(zbuf, 64)
    for k in range(ROWS_PER_TILE // 64):
        pltpu.sync_copy(zbuf, agg_sh.at[pl.ds(s * ROWS_PER_TILE + k * 64, 64)])
    plsc.subcore_barrier()

    def chunk(k):
        return jnp.minimum(wid + k * NW, NCHUNK)

    def gather(rows_slot, idx_slot):
        pltpu.async_copy(g_hbm.at[ebuf[idx_slot].at[0]], rows.at[rows_slot],
                         gsem[rows_slot])

    def scatter_start(rows_slot, idx_slot):
        pltpu.async_copy(rows.at[rows_slot], agg_sh.at[ebuf[idx_slot].at[1]],
                         ssem[rows_slot], add=True)

    def scatter_wait(rows_slot, idx_slot):
        pltpu.make_async_copy(rows.at[rows_slot],
                              agg_sh.at[ebuf[idx_slot].at[1]],
                              ssem[rows_slot]).wait()

    def iwait(idx_slot):
        pltpu.make_async_copy(eidx_hbm.at[0], ebuf[idx_slot],
                              isem[idx_slot]).wait()

    for b in range(2):
        pltpu.async_copy(eidx_hbm.at[chunk(b)], ebuf[b], isem[b])
    iwait(0)
    gather(0, 0)

    def step(k, phase, peeled_first=False):
        br = phase % 2
        brn = (br + 1) % 2
        bi = phase % 3
        bi1 = (phase + 1) % 3
        bi2 = (phase + 2) % 3
        iwait(bi1)
        if not peeled_first:
            scatter_wait(brn, bi2)
        gather(brn, bi1)
        pltpu.async_copy(eidx_hbm.at[chunk(k + 2)], ebuf[bi2], isem[bi2])
        pltpu.make_async_copy(
            g_hbm.at[ebuf[bi].at[0]], rows.at[br], gsem[br]).wait()
        scatter_start(br, bi)

    step(0, 0, peeled_first=True)

    def outer(i, _):
        for t in range(6):
            step(1 + i * 6 + t, 1 + t)
        return 0

    lax.fori_loop(0, (CPW - 2) // 6, outer, 0)
    step(CPW - 1, CPW - 1)
    scatter_wait((CPW - 1) % 2, (CPW - 1) % 3)
    pltpu.make_async_copy(
        g_hbm.at[ebuf[CPW % 3].at[0]], rows.at[CPW % 2], gsem[CPW % 2]).wait()
    iwait((CPW + 1) % 3)
    plsc.subcore_barrier()

    for k in range(ROWS_PER_TILE // 128):
        r0 = s * ROWS_PER_TILE + k * 128
        pltpu.sync_copy(agg_sh.at[pl.ds(r0, 128)],
                        out_hbm.at[c, pl.ds(r0, 128), :])


def _epi_body(aggp_ref, sfac_ref, msk_ref, cb_ref, bnw_ref, bnb_ref, out_ref):
    a = aggp_ref[0, :N, :] + aggp_ref[1, :N, :]
    out = a * sfac_ref[:N, :] + msk_ref[:N, :] * cb_ref[...]
    mean = jnp.mean(out, axis=0, keepdims=True)
    var = jnp.mean((out - mean) * (out - mean), axis=0, keepdims=True)
    y = (out - mean) / jnp.sqrt(var + EPS) * bnw_ref[...] + bnb_ref[...]
    out_ref[...] = (y / TAU >= V_TH).astype(jnp.float32)


_epilogue = pl.pallas_call(
    _epi_body,
    out_shape=jax.ShapeDtypeStruct((N, D), jnp.float32),
)


def kernel(x, edge_index, conv_w, conv_b, lin_res_w, lin_res_b, bn_w, bn_b):
    del lin_res_w, lin_res_b
    ei = edge_index.astype(jnp.int32)
    lanes = jnp.arange(CH, dtype=jnp.int32)[None, :]
    col_ext = jnp.concatenate(
        [ei[1].reshape(NCHUNK, CH), N + lanes], axis=0)
    eidx_ext = jnp.concatenate(
        [ei.reshape(2, NCHUNK, CH).transpose(1, 0, 2),
         jnp.stack([lanes, N + lanes], axis=1)], axis=0)

    degp = _deg_kernel(col_ext)
    degc = jnp.transpose(degp)
    g, sfac, msk = _proj(x, conv_w, degc)
    aggp = _scatter_kernel(g, eidx_ext)
    spike = _epilogue(aggp, sfac, msk,
                      conv_b.reshape(1, D),
                      bn_w.reshape(1, D), bn_b.reshape(1, D))
    return spike


# speedup vs baseline: 3.0313x; 1.0725x over previous
import functools

import jax
import jax.numpy as jnp
from jax import lax
from jax.experimental import pallas as pl
from jax.experimental.pallas import tpu as pltpu
from jax.experimental.pallas import tpu_sc as plsc

N = 10000
E = 320000
D = 128
NPAD = 10240
CH = 128
NCHUNK = E // CH
NW = 32
CPW = 80
ROWS_PER_TILE = NPAD // 16
TAU = 2.0
V_TH = 1.0
EPS = 1e-5

_mesh = plsc.VectorSubcoreMesh(
    core_axis_name="c", subcore_axis_name="s", num_cores=2, num_subcores=16)


def _zero_vmem_2d(ref, nrows):
    z = jnp.zeros((16,), jnp.float32)

    def body(i, _):
        for m in range(8):
            ref[i, pl.ds(m * 16, 16)] = z
        return 0

    lax.fori_loop(0, nrows, body, 0)


def _zero_vmem_1d(ref, n):
    z = jnp.zeros((16,), jnp.float32)

    def body(i, _):
        ref[pl.ds(i * 16, 16)] = z
        return 0

    lax.fori_loop(0, n // 16, body, 0)


@functools.partial(
    pl.kernel,
    out_type=jax.ShapeDtypeStruct((2, NPAD), jnp.float32),
    mesh=_mesh,
    scratch_types=[
        pltpu.VMEM((4, CH), jnp.int32),
        pltpu.VMEM((CH,), jnp.float32),
        pltpu.VMEM((ROWS_PER_TILE,), jnp.float32),
        pltpu.VMEM_SHARED((NPAD,), jnp.float32),
        pltpu.SemaphoreType.DMA,
        pltpu.SemaphoreType.DMA,
        pltpu.SemaphoreType.DMA,
        pltpu.SemaphoreType.DMA,
        pltpu.SemaphoreType.DMA,
        pltpu.SemaphoreType.DMA,
        pltpu.SemaphoreType.DMA,
        pltpu.SemaphoreType.DMA,
    ],
)
def _deg_kernel(eidx_hbm, dummy_hbm, out_hbm, cbuf, ones_v, zbuf, hist_sh,
                isem0, isem1, isem2, isem3, ssem0, ssem1, ssem2, ssem3):
    c = lax.axis_index("c")
    s = lax.axis_index("s")
    wid = s * 2 + c
    isem = (isem0, isem1, isem2, isem3)
    ssem = (ssem0, ssem1, ssem2, ssem3)

    _zero_vmem_1d(zbuf, ROWS_PER_TILE)
    one = jnp.ones((16,), jnp.float32)
    for m in range(CH // 16):
        ones_v[pl.ds(m * 16, 16)] = one
    pltpu.sync_copy(zbuf, hist_sh.at[pl.ds(s * ROWS_PER_TILE, ROWS_PER_TILE)])
    plsc.subcore_barrier()

    def prefetch(k, b):
        j = wid + k * NW

        @pl.when(j < NCHUNK)
        def _():
            pltpu.async_copy(eidx_hbm.at[1, pl.ds(j * CH, CH)],
                             cbuf.at[b], isem[b])

        @pl.when(j >= NCHUNK)
        def _():
            pltpu.async_copy(dummy_hbm.at[1], cbuf.at[b], isem[b])

    def swait(b):
        pltpu.make_async_copy(ones_v, hist_sh.at[cbuf.at[b]], ssem[b]).wait()

    prefetch(0, 0)
    prefetch(1, 1)

    def step(k, phase, first=False):
        b = phase % 4
        b2 = (phase + 2) % 4
        pltpu.make_async_copy(dummy_hbm.at[1], cbuf.at[b], isem[b]).wait()
        pltpu.async_copy(ones_v, hist_sh.at[cbuf.at[b]], ssem[b], add=True)
        if not first:
            swait(b2)
        prefetch(k + 2, b2)

    step(0, 0, first=True)
    step(1, 1, first=True)

    def outer(i, _):
        for t in range(4):
            step(2 + i * 4 + t, 2 + t)
        return 0

    lax.fori_loop(0, (CPW - 4) // 4, outer, 0)
    step(CPW - 2, CPW - 2)
    step(CPW - 1, CPW - 1)
    swait((CPW - 2) % 4)
    swait((CPW - 1) % 4)
    pltpu.make_async_copy(
        dummy_hbm.at[1], cbuf.at[CPW % 4], isem[CPW % 4]).wait()
    pltpu.make_async_copy(
        dummy_hbm.at[1], cbuf.at[(CPW + 1) % 4], isem[(CPW + 1) % 4]).wait()

    plsc.subcore_barrier()
    pltpu.sync_copy(hist_sh.at[pl.ds(s * ROWS_PER_TILE, ROWS_PER_TILE)],
                    out_hbm.at[c, pl.ds(s * ROWS_PER_TILE, ROWS_PER_TILE)])


def _proj_body(x_ref, w_ref, degc_ref, g_ref, sfac_ref, msk_ref):
    deg = degc_ref[:, 0:1] + degc_ref[:, 1:2]
    dinv = jnp.where(deg > 0, 1.0 / jnp.sqrt(jnp.maximum(deg, 1e-12)), 0.0)
    sfac_ref[...] = dinv / jnp.maximum(deg, 1.0)
    msk_ref[...] = (deg > 0).astype(jnp.float32)
    h = lax.dot_general(x_ref[...], w_ref[...], (((1,), (1,)), ((), ())),
                        preferred_element_type=jnp.float32)
    g_ref[...] = h * dinv[:N, :]


_proj = pl.pallas_call(
    _proj_body,
    out_shape=(
        jax.ShapeDtypeStruct((N, D), jnp.float32),
        jax.ShapeDtypeStruct((NPAD, 1), jnp.float32),
        jax.ShapeDtypeStruct((NPAD, 1), jnp.float32),
    ),
)


@functools.partial(
    pl.kernel,
    out_type=jax.ShapeDtypeStruct((2, NPAD, D), jnp.float32),
    mesh=_mesh,
    scratch_types=[
        pltpu.VMEM((2, CH), jnp.int32),
        pltpu.VMEM((2, CH), jnp.int32),
        pltpu.VMEM((2, CH), jnp.int32),
        pltpu.VMEM((2, CH, D), jnp.float32),
        pltpu.VMEM((64, D), jnp.float32),
        pltpu.VMEM_SHARED((NPAD, D), jnp.float32),
        pltpu.SemaphoreType.DMA,
        pltpu.SemaphoreType.DMA,
        pltpu.SemaphoreType.DMA,
        pltpu.SemaphoreType.DMA,
        pltpu.SemaphoreType.DMA,
        pltpu.SemaphoreType.DMA,
        pltpu.SemaphoreType.DMA,
    ],
)
def _scatter_kernel(g_hbm, eidx_hbm, dummy_hbm, out_hbm,
                    ebuf0, ebuf1, ebuf2, rows, zbuf,
                    agg_sh, gsem0, gsem1, ssem0, ssem1, is0, is1, is2):
    c = lax.axis_index("c")
    s = lax.axis_index("s")
    wid = s * 2 + c
    ebuf = (ebuf0, ebuf1, ebuf2)
    gsem = (gsem0, gsem1)
    ssem = (ssem0, ssem1)
    isem = (is0, is1, is2)

    _zero_vmem_2d(zbuf, 64)
    for k in range(ROWS_PER_TILE // 64):
        pltpu.sync_copy(zbuf, agg_sh.at[pl.ds(s * ROWS_PER_TILE + k * 64, 64)])
    plsc.subcore_barrier()

    def prefetch(k, b):
        j = wid + k * NW

        @pl.when(j < NCHUNK)
        def _():
            pltpu.async_copy(eidx_hbm.at[0, pl.ds(j * CH, CH)],
                             ebuf[b].at[0], isem[b])
            pltpu.async_copy(eidx_hbm.at[1, pl.ds(j * CH, CH)],
                             ebuf[b].at[1], isem[b])

        @pl.when(j >= NCHUNK)
        def _():
            pltpu.async_copy(dummy_hbm.at[0], ebuf[b].at[0], isem[b])
            pltpu.async_copy(dummy_hbm.at[1], ebuf[b].at[1], isem[b])

    def gather(rows_slot, idx_slot):
        pltpu.async_copy(g_hbm.at[ebuf[idx_slot].at[0]], rows.at[rows_slot],
                         gsem[rows_slot])

    def scatter_start(rows_slot, idx_slot):
        pltpu.async_copy(rows.at[rows_slot], agg_sh.at[ebuf[idx_slot].at[1]],
                         ssem[rows_slot], add=True)

    def scatter_wait(rows_slot, idx_slot):
        pltpu.make_async_copy(rows.at[rows_slot],
                              agg_sh.at[ebuf[idx_slot].at[1]],
                              ssem[rows_slot]).wait()

    def iwait(idx_slot):
        pltpu.make_async_copy(dummy_hbm.at[0], ebuf[idx_slot].at[0],
                              isem[idx_slot]).wait()
        pltpu.make_async_copy(dummy_hbm.at[1], ebuf[idx_slot].at[1],
                              isem[idx_slot]).wait()

    for b in range(2):
        prefetch(b, b)
    iwait(0)
    gather(0, 0)

    def step(k, phase, peeled_first=False):
        br = phase % 2
        brn = (br + 1) % 2
        bi = phase % 3
        bi1 = (phase + 1) % 3
        bi2 = (phase + 2) % 3
        iwait(bi1)
        if not peeled_first:
            scatter_wait(brn, bi2)
        gather(brn, bi1)
        prefetch(k + 2, bi2)
        pltpu.make_async_copy(
            g_hbm.at[ebuf[bi].at[0]], rows.at[br], gsem[br]).wait()
        scatter_start(br, bi)

    step(0, 0, peeled_first=True)

    def outer(i, _):
        for t in range(6):
            step(1 + i * 6 + t, 1 + t)
        return 0

    lax.fori_loop(0, (CPW - 2) // 6, outer, 0)
    step(CPW - 1, CPW - 1)
    scatter_wait((CPW - 1) % 2, (CPW - 1) % 3)
    pltpu.make_async_copy(
        g_hbm.at[ebuf[CPW % 3].at[0]], rows.at[CPW % 2], gsem[CPW % 2]).wait()
    iwait((CPW + 1) % 3)
    plsc.subcore_barrier()

    for k in range(ROWS_PER_TILE // 128):
        r0 = s * ROWS_PER_TILE + k * 128
        pltpu.sync_copy(agg_sh.at[pl.ds(r0, 128)],
                        out_hbm.at[c, pl.ds(r0, 128), :])


def _epi_body(aggp_ref, sfac_ref, msk_ref, cb_ref, bnw_ref, bnb_ref, out_ref):
    a = aggp_ref[0, :N, :] + aggp_ref[1, :N, :]
    out = a * sfac_ref[:N, :] + msk_ref[:N, :] * cb_ref[...]
    mean = jnp.mean(out, axis=0, keepdims=True)
    var = jnp.mean((out - mean) * (out - mean), axis=0, keepdims=True)
    y = (out - mean) / jnp.sqrt(var + EPS) * bnw_ref[...] + bnb_ref[...]
    out_ref[...] = (y / TAU >= V_TH).astype(jnp.float32)


_epilogue = pl.pallas_call(
    _epi_body,
    out_shape=jax.ShapeDtypeStruct((N, D), jnp.float32),
)


def kernel(x, edge_index, conv_w, conv_b, lin_res_w, lin_res_b, bn_w, bn_b):
    del lin_res_w, lin_res_b
    ei = edge_index.astype(jnp.int32)
    lanes = jnp.arange(CH, dtype=jnp.int32)
    dummy = jnp.stack([lanes, N + lanes], axis=0)

    degp = _deg_kernel(ei, dummy)
    degc = jnp.transpose(degp)
    g, sfac, msk = _proj(x, conv_w, degc)
    aggp = _scatter_kernel(g, ei, dummy)
    spike = _epilogue(aggp, sfac, msk,
                      conv_b.reshape(1, D),
                      bn_w.reshape(1, D), bn_b.reshape(1, D))
    return spike
